# baseline (speedup 1.0000x reference)
import jax
import jax.numpy as jnp
from jax import lax
from jax.experimental import pallas as pl
from jax.experimental.pallas import tpu as pltpu
from jax.experimental.pallas import tpu_sc as plsc

K = 8192
B = 64
HW = 32
NPIX = HW * HW
L = 16
NC = 2
NS = 16
NG = B // L


TBLK = 2048


def _table_body(cbt_ref, wt_ref, b_ref, out_ref):
    logits = lax.dot_general(wt_ref[...], cbt_ref[...],
                             (((1,), (0,)), ((), ())),
                             preferred_element_type=jnp.float32)
    out_ref[0:1, :] = jax.nn.sigmoid(logits[0:1, :] + b_ref[0])
    out_ref[1:2, :] = jax.nn.sigmoid(logits[1:2, :] + b_ref[1])


def _gather_body(tbl_hbm, zt_hbm,
                 endo_hbm, nuc_hbm, means_hbm,
                 tbl_e_v, tbl_n_v, zrow_v, oute_v, outn_v,
                 partial_v, red_v, mean_v, shared, sem1, sem2, sem3):
    core = lax.axis_index("c")
    sid = lax.axis_index("s")
    h = sid * NC + core
    c1 = pltpu.async_copy(tbl_hbm.at[0], tbl_e_v, sem1)
    c2 = pltpu.async_copy(tbl_hbm.at[1], tbl_n_v, sem2)
    c3 = pltpu.async_copy(zt_hbm.at[h], zrow_v, sem3)
    c1.wait()
    c2.wait()
    c3.wait()

    def body(w, accs):
        accs = list(accs)
        for g in range(NG):
            idx = zrow_v[w, pl.ds(g * L, L)]
            e = plsc.load_gather(tbl_e_v, [idx])
            n = plsc.load_gather(tbl_n_v, [idx])
            oute_v[w, pl.ds(g * L, L)] = e
            outn_v[w, pl.ds(g * L, L)] = n
            accs[g] = accs[g] + e
            accs[NG + g] = accs[NG + g] + n
        return tuple(accs)

    zero = jnp.zeros((L,), jnp.float32)
    accs = lax.fori_loop(0, HW, body, (zero,) * (2 * NG), unroll=4)
    o1 = pltpu.async_copy(oute_v, endo_hbm.at[h], sem1)
    o2 = pltpu.async_copy(outn_v, nuc_hbm.at[h], sem2)
    for g in range(2 * NG):
        partial_v[pl.ds(g * L, L)] = accs[g] * (1.0 / NPIX)
    pltpu.sync_copy(partial_v, shared.at[sid])
    plsc.subcore_barrier()

    @pl.when(sid == 0)
    def _():
        pltpu.sync_copy(shared, red_v)
        for g in range(2 * NG):
            acc = red_v[0, pl.ds(g * L, L)]
            for r in range(1, NS):
                acc = acc + red_v[r, pl.ds(g * L, L)]
            mean_v[pl.ds(g * L, L)] = acc
        pltpu.sync_copy(mean_v, means_hbm.at[core])

    o1.wait()
    o2.wait()


def kernel(z, codebook, lin_w, lin_b):
    tbl = pl.pallas_call(
        _table_body,
        grid=(K // TBLK,),
        out_shape=jax.ShapeDtypeStruct((2, K), jnp.float32),
        in_specs=[
            pl.BlockSpec((64, TBLK), lambda i: (0, i)),
            pl.BlockSpec((2, 64), lambda i: (0, 0)),
            pl.BlockSpec(memory_space=pltpu.SMEM),
        ],
        out_specs=pl.BlockSpec((2, TBLK), lambda i: (0, i)),
    )(codebook.T, lin_w.T, lin_b)
    zt = z.transpose(1, 2, 0).astype(jnp.int32)

    mesh = plsc.VectorSubcoreMesh(core_axis_name="c", subcore_axis_name="s")
    sc = pl.kernel(
        _gather_body,
        mesh=mesh,
        compiler_params=pltpu.CompilerParams(needs_layout_passes=False),
        out_type=[
            jax.ShapeDtypeStruct((HW, HW, B), jnp.float32),
            jax.ShapeDtypeStruct((HW, HW, B), jnp.float32),
            jax.ShapeDtypeStruct((NC, 2 * B), jnp.float32),
        ],
        scratch_types=[
            pltpu.VMEM((K,), jnp.float32),
            pltpu.VMEM((K,), jnp.float32),
            pltpu.VMEM((HW, B), jnp.int32),
            pltpu.VMEM((HW, B), jnp.float32),
            pltpu.VMEM((HW, B), jnp.float32),
            pltpu.VMEM((2 * B,), jnp.float32),
            pltpu.VMEM((NS, 2 * B), jnp.float32),
            pltpu.VMEM((2 * B,), jnp.float32),
            pltpu.VMEM_SHARED((NS, 2 * B), jnp.float32),
            pltpu.SemaphoreType.DMA,
            pltpu.SemaphoreType.DMA,
            pltpu.SemaphoreType.DMA,
        ],
    )
    oute, outn, means = sc(tbl, zt)
    endosome = oute.transpose(2, 0, 1).reshape(B, 1, HW, HW)
    nuclear = outn.transpose(2, 0, 1).reshape(B, 1, HW, HW)
    alea = (means[0, :B] + means[1, :B]).reshape(B, 1)
    epis = (means[0, B:] + means[1, B:]).reshape(B, 1)
    return (endosome, nuclear, alea, epis)


# speedup vs baseline: 1.0140x; 1.0140x over previous
import jax
import jax.numpy as jnp
from jax import lax
from jax.experimental import pallas as pl
from jax.experimental.pallas import tpu as pltpu
from jax.experimental.pallas import tpu_sc as plsc

K = 8192
B = 64
HW = 32
NPIX = HW * HW
L = 16
NC = 2
NS = 16
NG = B // L


TBLK = 2048


def _table_body(cbt_ref, wt_ref, b_ref, out_ref):
    logits = lax.dot_general(wt_ref[...], cbt_ref[...],
                             (((1,), (0,)), ((), ())),
                             preferred_element_type=jnp.float32)
    out_ref[0:1, :] = jax.nn.sigmoid(logits[0:1, :] + b_ref[0])
    out_ref[1:2, :] = jax.nn.sigmoid(logits[1:2, :] + b_ref[1])


def _gather_body(tbl_hbm, zt_hbm,
                 endo_hbm, nuc_hbm, means_hbm,
                 tbl_e_v, tbl_n_v, zrow_v, oute_v, outn_v,
                 partial_v, red_v, mean_v, shared, sem1, sem2, sem3):
    core = lax.axis_index("c")
    sid = lax.axis_index("s")
    h = sid * NC + core
    c1 = pltpu.async_copy(tbl_hbm.at[0], tbl_e_v, sem1)
    c2 = pltpu.async_copy(tbl_hbm.at[1], tbl_n_v, sem2)
    c3 = pltpu.async_copy(zt_hbm.at[h], zrow_v, sem3)
    c1.wait()
    c2.wait()
    c3.wait()

    def body(w, accs):
        accs = list(accs)
        for g in range(NG):
            idx = zrow_v[w, pl.ds(g * L, L)]
            e = plsc.load_gather(tbl_e_v, [idx])
            n = plsc.load_gather(tbl_n_v, [idx])
            oute_v[w, pl.ds(g * L, L)] = e
            outn_v[w, pl.ds(g * L, L)] = n
            accs[g] = accs[g] + e
            accs[NG + g] = accs[NG + g] + n
        return tuple(accs)

    zero = jnp.zeros((L,), jnp.float32)
    accs = lax.fori_loop(0, HW, body, (zero,) * (2 * NG), unroll=4)
    o1 = pltpu.async_copy(oute_v, endo_hbm.at[h], sem1)
    o2 = pltpu.async_copy(outn_v, nuc_hbm.at[h], sem2)
    for g in range(2 * NG):
        partial_v[pl.ds(g * L, L)] = accs[g] * (1.0 / NPIX)
    pltpu.sync_copy(partial_v, shared.at[sid])
    plsc.subcore_barrier()

    @pl.when(sid == 0)
    def _():
        pltpu.sync_copy(shared, red_v)
        for g in range(2 * NG):
            acc = red_v[0, pl.ds(g * L, L)]
            for r in range(1, NS):
                acc = acc + red_v[r, pl.ds(g * L, L)]
            mean_v[pl.ds(g * L, L)] = acc
        pltpu.sync_copy(mean_v, means_hbm.at[core])

    o1.wait()
    o2.wait()


def kernel(z, codebook, lin_w, lin_b):
    tbl = pl.pallas_call(
        _table_body,
        out_shape=jax.ShapeDtypeStruct((2, K), jnp.float32),
        in_specs=[
            pl.BlockSpec(memory_space=pltpu.VMEM),
            pl.BlockSpec(memory_space=pltpu.VMEM),
            pl.BlockSpec(memory_space=pltpu.SMEM),
        ],
    )(codebook.T, lin_w.T, lin_b)
    zt = z.transpose(1, 2, 0).astype(jnp.int32)

    mesh = plsc.VectorSubcoreMesh(core_axis_name="c", subcore_axis_name="s")
    sc = pl.kernel(
        _gather_body,
        mesh=mesh,
        compiler_params=pltpu.CompilerParams(needs_layout_passes=False),
        out_type=[
            jax.ShapeDtypeStruct((HW, HW, B), jnp.float32),
            jax.ShapeDtypeStruct((HW, HW, B), jnp.float32),
            jax.ShapeDtypeStruct((NC, 2 * B), jnp.float32),
        ],
        scratch_types=[
            pltpu.VMEM((K,), jnp.float32),
            pltpu.VMEM((K,), jnp.float32),
            pltpu.VMEM((HW, B), jnp.int32),
            pltpu.VMEM((HW, B), jnp.float32),
            pltpu.VMEM((HW, B), jnp.float32),
            pltpu.VMEM((2 * B,), jnp.float32),
            pltpu.VMEM((NS, 2 * B), jnp.float32),
            pltpu.VMEM((2 * B,), jnp.float32),
            pltpu.VMEM_SHARED((NS, 2 * B), jnp.float32),
            pltpu.SemaphoreType.DMA,
            pltpu.SemaphoreType.DMA,
            pltpu.SemaphoreType.DMA,
        ],
    )
    oute, outn, means = sc(tbl, zt)
    endosome = oute.transpose(2, 0, 1).reshape(B, 1, HW, HW)
    nuclear = outn.transpose(2, 0, 1).reshape(B, 1, HW, HW)
    alea = (means[0, :B] + means[1, :B]).reshape(B, 1)
    epis = (means[0, B:] + means[1, B:]).reshape(B, 1)
    return (endosome, nuclear, alea, epis)


# speedup vs baseline: 1.0338x; 1.0195x over previous
import jax
import jax.numpy as jnp
from jax import lax
from jax.experimental import pallas as pl
from jax.experimental.pallas import tpu as pltpu
from jax.experimental.pallas import tpu_sc as plsc

K = 8192
B = 64
HW = 32
NPIX = HW * HW
L = 16
NC = 2
NS = 16
NG = B // L


TBLK = 2048


def _table_body(cbt_ref, wt_ref, b_ref, out_ref):
    logits = lax.dot_general(wt_ref[...], cbt_ref[...],
                             (((1,), (0,)), ((), ())),
                             preferred_element_type=jnp.float32)
    out_ref[0:1, :] = jax.nn.sigmoid(logits[0:1, :] + b_ref[0])
    out_ref[1:2, :] = jax.nn.sigmoid(logits[1:2, :] + b_ref[1])


def _gather_body(tbl_hbm, zt_hbm,
                 endo_hbm, nuc_hbm, means_hbm,
                 tbl_e_v, tbl_n_v, zrow_v, oute_v, outn_v,
                 partial_v, red_v, mean_v, shared, sem1, sem2, sem3):
    core = lax.axis_index("c")
    sid = lax.axis_index("s")
    h = sid * NC + core
    c1 = pltpu.async_copy(tbl_hbm.at[0], tbl_e_v, sem1)
    c2 = pltpu.async_copy(tbl_hbm.at[1], tbl_n_v, sem2)
    c3 = pltpu.async_copy(zt_hbm.at[h], zrow_v, sem3)
    c1.wait()
    c2.wait()
    c3.wait()

    def body(w, accs):
        accs = list(accs)
        for g in range(NG):
            idx = zrow_v[w, pl.ds(g * L, L)]
            e = plsc.load_gather(tbl_e_v, [idx])
            n = plsc.load_gather(tbl_n_v, [idx])
            oute_v[w, pl.ds(g * L, L)] = e
            outn_v[w, pl.ds(g * L, L)] = n
            accs[g] = accs[g] + e
            accs[NG + g] = accs[NG + g] + n
        return tuple(accs)

    zero = jnp.zeros((L,), jnp.float32)
    accs = lax.fori_loop(0, HW, body, (zero,) * (2 * NG))
    o1 = pltpu.async_copy(oute_v, endo_hbm.at[h], sem1)
    o2 = pltpu.async_copy(outn_v, nuc_hbm.at[h], sem2)
    for g in range(2 * NG):
        partial_v[pl.ds(g * L, L)] = accs[g] * (1.0 / NPIX)
    pltpu.sync_copy(partial_v, shared.at[sid])
    plsc.subcore_barrier()

    @pl.when(sid == 0)
    def _():
        pltpu.sync_copy(shared, red_v)
        for g in range(2 * NG):
            acc = red_v[0, pl.ds(g * L, L)]
            for r in range(1, NS):
                acc = acc + red_v[r, pl.ds(g * L, L)]
            mean_v[pl.ds(g * L, L)] = acc
        pltpu.sync_copy(mean_v, means_hbm.at[core])

    o1.wait()
    o2.wait()


def kernel(z, codebook, lin_w, lin_b):
    tbl = pl.pallas_call(
        _table_body,
        out_shape=jax.ShapeDtypeStruct((2, K), jnp.float32),
        in_specs=[
            pl.BlockSpec(memory_space=pltpu.VMEM),
            pl.BlockSpec(memory_space=pltpu.VMEM),
            pl.BlockSpec(memory_space=pltpu.SMEM),
        ],
    )(codebook.T, lin_w.T, lin_b)
    zt = z.transpose(1, 2, 0).astype(jnp.int32)

    mesh = plsc.VectorSubcoreMesh(core_axis_name="c", subcore_axis_name="s")
    sc = pl.kernel(
        _gather_body,
        mesh=mesh,
        compiler_params=pltpu.CompilerParams(needs_layout_passes=False),
        out_type=[
            jax.ShapeDtypeStruct((HW, HW, B), jnp.float32),
            jax.ShapeDtypeStruct((HW, HW, B), jnp.float32),
            jax.ShapeDtypeStruct((NC, 2 * B), jnp.float32),
        ],
        scratch_types=[
            pltpu.VMEM((K,), jnp.float32),
            pltpu.VMEM((K,), jnp.float32),
            pltpu.VMEM((HW, B), jnp.int32),
            pltpu.VMEM((HW, B), jnp.float32),
            pltpu.VMEM((HW, B), jnp.float32),
            pltpu.VMEM((2 * B,), jnp.float32),
            pltpu.VMEM((NS, 2 * B), jnp.float32),
            pltpu.VMEM((2 * B,), jnp.float32),
            pltpu.VMEM_SHARED((NS, 2 * B), jnp.float32),
            pltpu.SemaphoreType.DMA,
            pltpu.SemaphoreType.DMA,
            pltpu.SemaphoreType.DMA,
        ],
    )
    oute, outn, means = sc(tbl, zt)
    endosome = oute.transpose(2, 0, 1).reshape(B, 1, HW, HW)
    nuclear = outn.transpose(2, 0, 1).reshape(B, 1, HW, HW)
    alea = (means[0, :B] + means[1, :B]).reshape(B, 1)
    epis = (means[0, B:] + means[1, B:]).reshape(B, 1)
    return (endosome, nuclear, alea, epis)


# speedup vs baseline: 1.0611x; 1.0264x over previous
import jax
import jax.numpy as jnp
from jax import lax
from jax.experimental import pallas as pl
from jax.experimental.pallas import tpu as pltpu
from jax.experimental.pallas import tpu_sc as plsc

K = 8192
B = 64
HW = 32
NPIX = HW * HW
L = 16
NC = 2
NS = 16
NG = B // L


TBLK = 2048


def _table_body(cbt_ref, wt_ref, b_ref, out_ref):
    logits = lax.dot_general(wt_ref[...], cbt_ref[...],
                             (((1,), (0,)), ((), ())),
                             preferred_element_type=jnp.float32)
    se = jax.nn.sigmoid(logits[0:1, :] + b_ref[0])
    sn = jax.nn.sigmoid(logits[1:2, :] + b_ref[1])
    eu = lax.bitcast_convert_type(se.astype(jnp.bfloat16),
                                  jnp.uint16).astype(jnp.uint32)
    nu = lax.bitcast_convert_type(sn.astype(jnp.bfloat16),
                                  jnp.uint16).astype(jnp.uint32)
    out_ref[...] = lax.bitcast_convert_type(eu | (nu << 16),
                                            jnp.int32).reshape(K)


def _gather_body(tbl_hbm, zt_hbm,
                 endo_hbm, nuc_hbm, means_hbm,
                 tbl_v, zrow_v, oute_v, outn_v,
                 partial_v, red_v, mean_v, shared, sem1, sem2, sem3):
    core = lax.axis_index("c")
    sid = lax.axis_index("s")
    h = sid * NC + core
    c1 = pltpu.async_copy(tbl_hbm, tbl_v, sem1)
    c3 = pltpu.async_copy(zt_hbm.at[h], zrow_v, sem3)
    c1.wait()
    c3.wait()
    hi_mask = jnp.full((L,), -65536, jnp.int32)

    def body(w, accs):
        accs = list(accs)
        for g in range(NG):
            idx = zrow_v[w, pl.ds(g * L, L)]
            u = plsc.load_gather(tbl_v, [idx])
            e = plsc.bitcast(u << 16, jnp.float32)
            n = plsc.bitcast(u & hi_mask, jnp.float32)
            oute_v[w, pl.ds(g * L, L)] = e
            outn_v[w, pl.ds(g * L, L)] = n
            accs[g] = accs[g] + e
            accs[NG + g] = accs[NG + g] + n
        return tuple(accs)

    zero = jnp.zeros((L,), jnp.float32)
    accs = lax.fori_loop(0, HW, body, (zero,) * (2 * NG))
    o1 = pltpu.async_copy(oute_v, endo_hbm.at[h], sem1)
    o2 = pltpu.async_copy(outn_v, nuc_hbm.at[h], sem2)
    for g in range(2 * NG):
        partial_v[pl.ds(g * L, L)] = accs[g] * (1.0 / NPIX)
    pltpu.sync_copy(partial_v, shared.at[sid])
    plsc.subcore_barrier()

    @pl.when(sid == 0)
    def _():
        pltpu.sync_copy(shared, red_v)
        for g in range(2 * NG):
            acc = red_v[0, pl.ds(g * L, L)]
            for r in range(1, NS):
                acc = acc + red_v[r, pl.ds(g * L, L)]
            mean_v[pl.ds(g * L, L)] = acc
        pltpu.sync_copy(mean_v, means_hbm.at[core])

    o1.wait()
    o2.wait()


def kernel(z, codebook, lin_w, lin_b):
    tbl = pl.pallas_call(
        _table_body,
        out_shape=jax.ShapeDtypeStruct((K,), jnp.int32),
        in_specs=[
            pl.BlockSpec(memory_space=pltpu.VMEM),
            pl.BlockSpec(memory_space=pltpu.VMEM),
            pl.BlockSpec(memory_space=pltpu.SMEM),
        ],
    )(codebook.T, lin_w.T, lin_b)
    zt = z.transpose(1, 2, 0).astype(jnp.int32)

    mesh = plsc.VectorSubcoreMesh(core_axis_name="c", subcore_axis_name="s")
    sc = pl.kernel(
        _gather_body,
        mesh=mesh,
        compiler_params=pltpu.CompilerParams(needs_layout_passes=False),
        out_type=[
            jax.ShapeDtypeStruct((HW, HW, B), jnp.float32),
            jax.ShapeDtypeStruct((HW, HW, B), jnp.float32),
            jax.ShapeDtypeStruct((NC, 2 * B), jnp.float32),
        ],
        scratch_types=[
            pltpu.VMEM((K,), jnp.int32),
            pltpu.VMEM((HW, B), jnp.int32),
            pltpu.VMEM((HW, B), jnp.float32),
            pltpu.VMEM((HW, B), jnp.float32),
            pltpu.VMEM((2 * B,), jnp.float32),
            pltpu.VMEM((NS, 2 * B), jnp.float32),
            pltpu.VMEM((2 * B,), jnp.float32),
            pltpu.VMEM_SHARED((NS, 2 * B), jnp.float32),
            pltpu.SemaphoreType.DMA,
            pltpu.SemaphoreType.DMA,
            pltpu.SemaphoreType.DMA,
        ],
    )
    oute, outn, means = sc(tbl, zt)
    endosome = oute.transpose(2, 0, 1).reshape(B, 1, HW, HW)
    nuclear = outn.transpose(2, 0, 1).reshape(B, 1, HW, HW)
    alea = (means[0, :B] + means[1, :B]).reshape(B, 1)
    epis = (means[0, B:] + means[1, B:]).reshape(B, 1)
    return (endosome, nuclear, alea, epis)


# speedup vs baseline: 1.0806x; 1.0184x over previous
import jax
import jax.numpy as jnp
from jax import lax
from jax.experimental import pallas as pl
from jax.experimental.pallas import tpu as pltpu
from jax.experimental.pallas import tpu_sc as plsc

K = 8192
B = 64
HW = 32
NPIX = HW * HW
L = 16
NC = 2
NS = 16
NG = B // L


TBLK = 2048


def _table_body(cbt_ref, wt_ref, b_ref, out_ref):
    logits = lax.dot_general(wt_ref[...], cbt_ref[...],
                             (((1,), (0,)), ((), ())),
                             preferred_element_type=jnp.float32)
    se = jax.nn.sigmoid(logits[0:1, :] + b_ref[0])
    sn = jax.nn.sigmoid(logits[1:2, :] + b_ref[1])
    eu = lax.bitcast_convert_type(se.astype(jnp.bfloat16),
                                  jnp.uint16).astype(jnp.uint32)
    nu = lax.bitcast_convert_type(sn.astype(jnp.bfloat16),
                                  jnp.uint16).astype(jnp.uint32)
    out_ref[...] = lax.bitcast_convert_type(eu | (nu << 16),
                                            jnp.int32).reshape(K)


def _gather_body(tbl_hbm, zt_hbm,
                 endo_hbm, nuc_hbm, means_hbm,
                 tbl_v, zrow_v, oute_v, outn_v,
                 partial_v, red_v, mean_v, shared, sem1, sem2, sem3):
    core = lax.axis_index("c")
    sid = lax.axis_index("s")
    h = sid * NC + core
    c1 = pltpu.async_copy(tbl_hbm, tbl_v, sem1)
    c3 = pltpu.async_copy(zt_hbm.at[h], zrow_v, sem3)
    c1.wait()
    c3.wait()
    hi_mask = jnp.full((L,), -65536, jnp.int32)

    def body(w, accs):
        accs = list(accs)
        for g in range(NG):
            idx = zrow_v[w, pl.ds(g * L, L)]
            u = plsc.load_gather(tbl_v, [idx])
            e = plsc.bitcast(u << 16, jnp.float32)
            n = plsc.bitcast(u & hi_mask, jnp.float32)
            oute_v[w, pl.ds(g * L, L)] = e
            outn_v[w, pl.ds(g * L, L)] = n
            accs[g] = accs[g] + e
            accs[NG + g] = accs[NG + g] + n
        return tuple(accs)

    zero = jnp.zeros((L,), jnp.float32)
    accs = lax.fori_loop(0, HW, body, (zero,) * (2 * NG))
    o1 = pltpu.async_copy(oute_v, endo_hbm.at[h], sem1)
    o2 = pltpu.async_copy(outn_v, nuc_hbm.at[h], sem2)
    for g in range(2 * NG):
        partial_v[pl.ds(g * L, L)] = accs[g] * (1.0 / NPIX)
    pltpu.sync_copy(partial_v, shared.at[sid])
    plsc.subcore_barrier()

    @pl.when(sid < 2 * NG)
    def _():
        pltpu.sync_copy(shared, red_v)
        g0 = sid * L
        acc = red_v[0, pl.ds(g0, L)]
        for r in range(1, NS):
            acc = acc + red_v[r, pl.ds(g0, L)]
        mean_v[pl.ds(0, L)] = acc
        pltpu.sync_copy(mean_v.at[pl.ds(0, L)],
                        means_hbm.at[pl.ds(core * 2 * B + g0, L)])

    o1.wait()
    o2.wait()


def kernel(z, codebook, lin_w, lin_b):
    tbl = pl.pallas_call(
        _table_body,
        out_shape=jax.ShapeDtypeStruct((K,), jnp.int32),
        in_specs=[
            pl.BlockSpec(memory_space=pltpu.VMEM),
            pl.BlockSpec(memory_space=pltpu.VMEM),
            pl.BlockSpec(memory_space=pltpu.SMEM),
        ],
    )(codebook.T, lin_w.T, lin_b)
    zt = z.transpose(1, 2, 0).astype(jnp.int32)

    mesh = plsc.VectorSubcoreMesh(core_axis_name="c", subcore_axis_name="s")
    sc = pl.kernel(
        _gather_body,
        mesh=mesh,
        compiler_params=pltpu.CompilerParams(needs_layout_passes=False),
        out_type=[
            jax.ShapeDtypeStruct((HW, HW, B), jnp.float32),
            jax.ShapeDtypeStruct((HW, HW, B), jnp.float32),
            jax.ShapeDtypeStruct((NC * 2 * B,), jnp.float32),
        ],
        scratch_types=[
            pltpu.VMEM((K,), jnp.int32),
            pltpu.VMEM((HW, B), jnp.int32),
            pltpu.VMEM((HW, B), jnp.float32),
            pltpu.VMEM((HW, B), jnp.float32),
            pltpu.VMEM((2 * B,), jnp.float32),
            pltpu.VMEM((NS, 2 * B), jnp.float32),
            pltpu.VMEM((2 * B,), jnp.float32),
            pltpu.VMEM_SHARED((NS, 2 * B), jnp.float32),
            pltpu.SemaphoreType.DMA,
            pltpu.SemaphoreType.DMA,
            pltpu.SemaphoreType.DMA,
        ],
    )
    oute, outn, means = sc(tbl, zt)
    endosome = oute.transpose(2, 0, 1).reshape(B, 1, HW, HW)
    nuclear = outn.transpose(2, 0, 1).reshape(B, 1, HW, HW)
    alea = (means[:B] + means[2 * B:3 * B]).reshape(B, 1)
    epis = (means[B:2 * B] + means[3 * B:]).reshape(B, 1)
    return (endosome, nuclear, alea, epis)
